# baseline (speedup 1.0000x reference)
import functools

import jax
import jax.numpy as jnp
from jax import lax
from jax.experimental import pallas as pl
from jax.experimental.pallas import tpu as pltpu
from jax.experimental.pallas import tpu_sc as plsc

EMBED_DIM = 128
NUM_CORES = 2
NUM_SUBCORES = 16
NUM_WORKERS = NUM_CORES * NUM_SUBCORES
CHUNK = 128


def _make_emb_kernel(total_indices: int):
  per_worker = total_indices // NUM_WORKERS
  n_chunks = per_worker // CHUNK
  mesh = plsc.VectorSubcoreMesh(
      core_axis_name="c", subcore_axis_name="s",
      num_cores=NUM_CORES, num_subcores=NUM_SUBCORES)

  nbuf = 4
  assert n_chunks % nbuf == 0 and n_chunks >= 2 * nbuf

  @functools.partial(
      pl.kernel,
      out_type=jax.ShapeDtypeStruct((total_indices, EMBED_DIM), jnp.float32),
      mesh=mesh,
      scratch_types=[
          pltpu.VMEM((n_chunks, CHUNK), jnp.int32),
          [pltpu.VMEM((CHUNK, EMBED_DIM), jnp.float32) for _ in range(nbuf)],
          [pltpu.SemaphoreType.DMA for _ in range(nbuf)],
          [pltpu.SemaphoreType.DMA for _ in range(nbuf)],
      ],
  )
  def emb_kernel(table_hbm, idx_hbm, out_hbm, idx_v, bufs, sem_in, sem_out):
    wid = lax.axis_index("s") * NUM_CORES + lax.axis_index("c")
    base = wid * per_worker
    pltpu.sync_copy(idx_hbm.at[wid], idx_v)

    def gather(j, b):
      pltpu.async_copy(table_hbm.at[idx_v.at[j]], bufs[b], sem_in[b])

    def gather_wait(j, b):
      pltpu.make_async_copy(table_hbm.at[idx_v.at[j]], bufs[b],
                            sem_in[b]).wait()

    def out_start(j, b):
      pltpu.async_copy(bufs[b], out_hbm.at[pl.ds(base + j * CHUNK, CHUNK)],
                       sem_out[b])

    def out_wait(j, b):
      pltpu.make_async_copy(bufs[b],
                            out_hbm.at[pl.ds(base + j * CHUNK, CHUNK)],
                            sem_out[b]).wait()

    for b in range(nbuf):
      gather(b, b)

    def body(i, carry):
      g = i * nbuf
      for b in range(nbuf):
        gather_wait(g + b, b)
        out_start(g + b, b)
      for b in range(nbuf):
        out_wait(g + b, b)
        gather(g + nbuf + b, b)
      return carry

    lax.fori_loop(0, n_chunks // nbuf - 1, body, 0)
    g = n_chunks - nbuf
    for b in range(nbuf):
      gather_wait(g + b, b)
      out_start(g + b, b)
    for b in range(nbuf):
      out_wait(g + b, b)

  return emb_kernel


def kernel(inputs, table):
  batch, hist = inputs.shape
  total = batch * hist
  idx = inputs.astype(jnp.int32).reshape(
      NUM_WORKERS, total // (NUM_WORKERS * CHUNK), CHUNK)
  out = _make_emb_kernel(total)(table, idx)
  return out.reshape(batch, hist, EMBED_DIM)
